# Initial kernel scaffold; baseline (speedup 1.0000x reference)
import jax
import jax.numpy as jnp
from jax.experimental import pallas as pl


def kernel(x_QENT, x_CENT, x_SPAN, x_SENT, ei_qent_span, ei_qent_sent, ei_cent_sent, ei_span_cent, ei_cent_cooccur, params):
    raise NotImplementedError("write your pallas kernel here")


# speedup vs baseline: 7.7143x; 7.7143x over previous
import functools

import jax
import jax.numpy as jnp
from jax import lax
from jax.experimental import pallas as pl
from jax.experimental.pallas import tpu as pltpu
from jax.experimental.pallas import tpu_sc as plsc

HID = 128
N_SRC = 10000
N_DST = 50000
E = 100000

NSUB = 16
NCORE = 2
SLAB = 10240
NSLABS = 5
NPASS = 3
OUT_ROWS = SLAB * NSLABS
CROWS = 56
CHUNK = CROWS * 128
E_PAD = CHUNK * NSUB
BR = 8
NB = CROWS // BR
AD_PAD = SLAB * (NSLABS + 1)
SRC_PAD = 10240
CAP = 2048
CAP_WM = CAP - 16
KC = 64
ZCH = 64
NZO = 10
STRIPE = SLAB // NSUB
DCH = SLAB // 4
DST_PAD_IDX = SLAB * (NSLABS + 1)


def _sc_edge_body(hs_hbm, a_s_hbm, a_d_hbm, src_hbm, dst_hbm, g_hbm,
                  acc_out, den_out,
                  acc_sp, den_sp,
                  a_s_v, ad_slab, src_b0, dst_b0, src_b1, dst_b1,
                  cbuf_s, cbuf_d, rows_a, rows_b,
                  xst_a, xst_b, ldx_a, ldx_b, dbounce, gv,
                  sem_ga, sem_gb, sem_sa, sem_sb, sem_e, sem_z):
    c = lax.axis_index("c")
    s = lax.axis_index("s")

    pltpu.sync_copy(a_s_hbm, a_s_v)
    pltpu.sync_copy(g_hbm, gv)
    gval = gv[...]

    z16f = jnp.zeros((16,), jnp.float32)
    z16i = jnp.zeros((16,), jnp.int32)
    lane16 = lax.iota(jnp.int32, 16)

    def _zcb(i, _):
        cbuf_s[pl.ds(i * 16, 16)] = z16i
        cbuf_d[pl.ds(i * 16, 16)] = z16i
        return 0
    lax.fori_loop(0, CAP // 16, _zcb, 0)

    def gissue(k, rows, semg):
        pltpu.async_copy(hs_hbm.at[cbuf_s.at[pl.ds(k * KC, KC)]],
                         rows, semg)

    def gwait(k, rows, semg):
        pltpu.make_async_copy(hs_hbm.at[cbuf_s.at[pl.ds(k * KC, KC)]],
                              rows, semg).wait()

    def drain(ncomp):
        nch = (ncomp + KC - 1) // KC

        def ex_into(k, xst, ldx):
            rem = ncomp - k * KC
            for j in range(KC // 16):
                sj = cbuf_s[pl.ds(k * KC + j * 16, 16)]
                lj = cbuf_d[pl.ds(k * KC + j * 16, 16)]
                es = plsc.load_gather(a_s_v, [sj])
                ed = plsc.load_gather(ad_slab, [lj])
                x_ = es + ed
                e_ = jnp.where(x_ >= 0.0, x_, x_ * 0.2)
                exv = jnp.exp(e_ - gval)
                exv = jnp.where(j * 16 + lane16 < rem, exv, 0.0)
                xst[pl.ds(j * 16, 16)] = exv
                ldx[pl.ds(j * 16, 16)] = lj

        def scale(rows, xst):
            def _s(r, _):
                a = plsc.load_gather(xst, [jnp.full((16,), r, jnp.int32)])
                row = rows.at[r]
                for j in range(8):
                    row[pl.ds(j * 16, 16)] = row[pl.ds(j * 16, 16)] * a
                return 0
            lax.fori_loop(0, KC, _s, 0)

        def scatter(rows, xst, ldx):
            pltpu.sync_copy(rows, acc_sp.at[ldx], add=True)
            pltpu.sync_copy(xst, den_sp.at[ldx], add=True)

        gissue(0, rows_a, sem_ga)

        def step(t, _):
            ka = 2 * t
            kb = 2 * t + 1
            gwait(ka, rows_a, sem_ga)

            @pl.when(kb < nch)
            def _():
                gissue(kb, rows_b, sem_gb)
            ex_into(ka, xst_a, ldx_a)
            scale(rows_a, xst_a)
            scatter(rows_a, xst_a, ldx_a)

            @pl.when(kb < nch)
            def _():
                gwait(kb, rows_b, sem_gb)

                @pl.when(kb + 1 < nch)
                def _():
                    gissue(kb + 1, rows_a, sem_ga)
                ex_into(kb, xst_b, ldx_b)
                scale(rows_b, xst_b)
                scatter(rows_b, xst_b, ldx_b)
            return 0

        lax.fori_loop(0, (nch + 1) // 2, step, 0)

    for p_ in range(NPASS):
        slab_i = NCORE * p_ + c
        base = slab_i * SLAB

        pltpu.async_copy(
            a_d_hbm.at[pl.ds(pl.multiple_of(base, 128), SLAB)], ad_slab,
            sem_e)

        def _zero_rows(r, _):
            row = rows_a.at[r]
            for j in range(8):
                row[pl.ds(j * 16, 16)] = z16f
            return 0
        lax.fori_loop(0, ZCH, _zero_rows, 0)

        zsrc = rows_a.at[pl.ds(0, ZCH)]
        for i in range(NZO):
            row0 = pl.multiple_of(s * STRIPE + i * ZCH, 8)
            pltpu.async_copy(zsrc, acc_sp.at[pl.ds(row0, ZCH)], sem_z)

        @pl.when(s < 4)
        def _():
            def _zero_db(i, _):
                dbounce[pl.ds(i * 16, 16)] = z16f
                return 0
            lax.fori_loop(0, DCH // 16, _zero_db, 0)
            off = pl.multiple_of(s * DCH, 128)
            pltpu.sync_copy(dbounce, den_sp.at[pl.ds(off, DCH)])

        for i in range(NZO):
            row0 = pl.multiple_of(s * STRIPE + i * ZCH, 8)
            pltpu.make_async_copy(zsrc, acc_sp.at[pl.ds(row0, ZCH)],
                                  sem_z).wait()

        pltpu.make_async_copy(
            a_d_hbm.at[pl.ds(pl.multiple_of(base, 128), SLAB)], ad_slab,
            sem_e).wait()

        plsc.subcore_barrier()

        def _group_body(blk_src, blk_dst):
            def _group(g_, ptr):
                q = g_ // 8
                r = g_ % 8
                s16 = blk_src[q, pl.ds(r * 16, 16)]
                d16 = blk_dst[q, pl.ds(r * 16, 16)]
                m = (d16 >= base) & (d16 < base + SLAB)
                ld = d16 - base
                plsc.store_compressed(cbuf_s.at[pl.ds(ptr, 16)], s16, mask=m)
                plsc.store_compressed(cbuf_d.at[pl.ds(ptr, 16)], ld, mask=m)
                nptr = ptr + jnp.sum(m.astype(jnp.int32))
                do_flush = nptr > CAP_WM

                @pl.when(do_flush)
                def _():
                    drain(nptr)
                return jnp.where(do_flush, 0, nptr)
            return _group

        bufs = [(src_b0, dst_b0), (src_b1, dst_b1)]

        def bissue(b):
            pltpu.async_copy(src_hbm.at[s, pl.ds(b * BR, BR)],
                             bufs[b % 2][0], sem_sa)
            pltpu.async_copy(dst_hbm.at[s, pl.ds(b * BR, BR)],
                             bufs[b % 2][1], sem_sb)

        def bwait(b):
            pltpu.make_async_copy(src_hbm.at[s, pl.ds(b * BR, BR)],
                                  bufs[b % 2][0], sem_sa).wait()
            pltpu.make_async_copy(dst_hbm.at[s, pl.ds(b * BR, BR)],
                                  bufs[b % 2][1], sem_sb).wait()

        bissue(0)
        ptr = jnp.int32(0)
        for b in range(NB):
            bwait(b)
            if b + 1 < NB:
                bissue(b + 1)
            ptr = lax.fori_loop(0, BR * 8,
                                _group_body(bufs[b % 2][0], bufs[b % 2][1]),
                                ptr)

        @pl.when(ptr > 0)
        def _():
            drain(ptr)
        plsc.subcore_barrier()

        @pl.when(slab_i < NSLABS)
        def _():
            for i in range(NZO):
                row0 = pl.multiple_of(s * STRIPE + i * ZCH, 8)
                pltpu.async_copy(acc_sp.at[pl.ds(row0, ZCH)],
                                 acc_out.at[pl.ds(base + row0, ZCH)], sem_z)

            @pl.when(s < 4)
            def _():
                off = pl.multiple_of(s * DCH, 128)
                pltpu.sync_copy(den_sp.at[pl.ds(off, DCH)], dbounce)
                pltpu.sync_copy(dbounce, den_out.at[pl.ds(base + off, DCH)])

            for i in range(NZO):
                row0 = pl.multiple_of(s * STRIPE + i * ZCH, 8)
                pltpu.make_async_copy(acc_sp.at[pl.ds(row0, ZCH)],
                                      acc_out.at[pl.ds(base + row0, ZCH)],
                                      sem_z).wait()

        plsc.subcore_barrier()


_sc_edge = pl.kernel(
    _sc_edge_body,
    out_type=[
        jax.ShapeDtypeStruct((OUT_ROWS, HID), jnp.float32),
        jax.ShapeDtypeStruct((OUT_ROWS,), jnp.float32),
    ],
    mesh=plsc.VectorSubcoreMesh(core_axis_name="c", subcore_axis_name="s"),
    compiler_params=pltpu.CompilerParams(needs_layout_passes=False),
    scratch_types=[
        pltpu.VMEM_SHARED((SLAB, HID), jnp.float32),
        pltpu.VMEM_SHARED((SLAB,), jnp.float32),
        pltpu.VMEM((SRC_PAD,), jnp.float32),
        pltpu.VMEM((SLAB,), jnp.float32),
        pltpu.VMEM((BR, 128), jnp.int32),
        pltpu.VMEM((BR, 128), jnp.int32),
        pltpu.VMEM((BR, 128), jnp.int32),
        pltpu.VMEM((BR, 128), jnp.int32),
        pltpu.VMEM((CAP,), jnp.int32),
        pltpu.VMEM((CAP,), jnp.int32),
        pltpu.VMEM((KC, HID), jnp.float32),
        pltpu.VMEM((KC, HID), jnp.float32),
        pltpu.VMEM((KC,), jnp.float32),
        pltpu.VMEM((KC,), jnp.float32),
        pltpu.VMEM((KC,), jnp.int32),
        pltpu.VMEM((KC,), jnp.int32),
        pltpu.VMEM((DCH,), jnp.float32),
        pltpu.VMEM((16,), jnp.float32),
        pltpu.SemaphoreType.DMA,
        pltpu.SemaphoreType.DMA,
        pltpu.SemaphoreType.DMA,
        pltpu.SemaphoreType.DMA,
        pltpu.SemaphoreType.DMA,
        pltpu.SemaphoreType.DMA,
    ],
)


def _dense_q_body(x_ref, wlq_ref, blq_ref, w1_ref, a1_ref, w2_ref, a2_ref,
                  hs1_ref, hs2_ref, as1_ref, as2_ref):
    x = x_ref[...]
    h0 = jnp.dot(x, wlq_ref[...], preferred_element_type=jnp.float32) + blq_ref[...]
    hs1 = jnp.dot(h0, w1_ref[...], preferred_element_type=jnp.float32)
    hs1_ref[...] = hs1
    as1_ref[...] = jnp.sum(hs1 * a1_ref[...], axis=1).reshape(1, 1, 512)
    q = jnp.maximum(h0, 0.0)
    hs2 = jnp.dot(q, w2_ref[...], preferred_element_type=jnp.float32)
    hs2_ref[...] = hs2
    as2_ref[...] = jnp.sum(hs2 * a2_ref[...], axis=1).reshape(1, 1, 512)


def _dense_q(x_pad, wlq, blq, w1, a1, w2, a2):
    nb = SRC_PAD // 512
    mat = pl.BlockSpec((512, HID), lambda i: (i, 0))
    full = pl.BlockSpec((HID, HID), lambda i: (0, 0))
    vec = pl.BlockSpec((1, HID), lambda i: (0, 0))
    col = pl.BlockSpec((1, 1, 512), lambda i: (i, 0, 0))
    return pl.pallas_call(
        _dense_q_body,
        grid=(nb,),
        in_specs=[mat, full, vec, full, vec, full, vec],
        out_specs=[mat, mat, col, col],
        out_shape=[
            jax.ShapeDtypeStruct((SRC_PAD, HID), jnp.float32),
            jax.ShapeDtypeStruct((SRC_PAD, HID), jnp.float32),
            jax.ShapeDtypeStruct((nb, 1, 512), jnp.float32),
            jax.ShapeDtypeStruct((nb, 1, 512), jnp.float32),
        ],
    )(x_pad, wlq, blq, w1, a1, w2, a2)


def _matvec_body(x_ref, v_ref, c_ref, y_ref):
    y = jnp.sum(x_ref[...] * v_ref[...], axis=1) + c_ref[0, 0]
    y_ref[...] = y.reshape(1, 1, 512)


def _matvec(x_pad, v, cconst):
    n = x_pad.shape[0]
    nb = n // 512
    return pl.pallas_call(
        _matvec_body,
        grid=(nb,),
        in_specs=[pl.BlockSpec((512, HID), lambda i: (i, 0)),
                  pl.BlockSpec((1, HID), lambda i: (0, 0)),
                  pl.BlockSpec((1, 1), lambda i: (0, 0))],
        out_specs=pl.BlockSpec((1, 1, 512), lambda i: (i, 0, 0)),
        out_shape=jax.ShapeDtypeStruct((nb, 1, 512), jnp.float32),
    )(x_pad, v, cconst)


def _postmv_body(acc_ref, den_ref, bias_ref, v_ref, c_ref, y_ref):
    inv = 1.0 / (den_ref[0, 0, :] + 1e-16)
    xd = jnp.maximum(acc_ref[...] * inv[:, None] + bias_ref[...], 0.0)
    y = jnp.sum(xd * v_ref[...], axis=1) + c_ref[0, 0]
    y_ref[...] = y.reshape(1, 1, 512)


def _postmv(acc, den, bias, v, cconst):
    nb = OUT_ROWS // 512
    return pl.pallas_call(
        _postmv_body,
        grid=(nb,),
        in_specs=[pl.BlockSpec((512, HID), lambda i: (i, 0)),
                  pl.BlockSpec((1, 1, 512), lambda i: (i, 0, 0)),
                  pl.BlockSpec((1, HID), lambda i: (0, 0)),
                  pl.BlockSpec((1, HID), lambda i: (0, 0)),
                  pl.BlockSpec((1, 1), lambda i: (0, 0))],
        out_specs=pl.BlockSpec((1, 1, 512), lambda i: (i, 0, 0)),
        out_shape=jax.ShapeDtypeStruct((nb, 1, 512), jnp.float32),
    )(acc, den.reshape(OUT_ROWS // 512, 1, 512), bias, v, cconst)


def kernel(x_QENT, x_CENT, x_SPAN, x_SENT,
           ei_qent_span, ei_qent_sent, ei_cent_sent, ei_span_cent,
           ei_cent_cooccur, params):
    p1 = params["convs"][0]["qent_span"]
    p2 = params["convs"][1]["qent_span"]
    lq = params["lin"]["QENT"]
    ls = params["lin"]["SPAN"]

    f32 = jnp.float32
    xq_pad = jnp.pad(x_QENT.astype(f32), ((0, SRC_PAD - N_SRC), (0, 0)))
    xs_pad = jnp.pad(x_SPAN.astype(f32), ((0, 50176 - N_DST), (0, 0)))

    w1d = p1["W_dst"] @ p1["a_dst"]
    v1 = (ls["W"] @ w1d).reshape(1, HID)
    c1 = jnp.dot(ls["b"], w1d).reshape(1, 1)
    w2d = (p2["W_dst"] @ p2["a_dst"]).reshape(1, HID)

    hs1, hs2, as1_3d, as2_3d = _dense_q(
        xq_pad, lq["W"], lq["b"].reshape(1, HID),
        p1["W_src"], p1["a_src"].reshape(1, HID),
        p2["W_src"], p2["a_src"].reshape(1, HID))
    as1 = as1_3d.reshape(SRC_PAD)
    as2 = as2_3d.reshape(SRC_PAD)

    ad1 = _matvec(xs_pad, v1, c1).reshape(50176)

    src = ei_qent_span[0]
    dst = ei_qent_span[1]
    src_p = jnp.pad(src, (0, E_PAD - E)).reshape(NSUB, CROWS, 128)
    dst_p = jnp.pad(dst, (0, E_PAD - E),
                    constant_values=DST_PAD_IDX).reshape(NSUB, CROWS, 128)
    as1_sc = jnp.pad(as1[:N_SRC], (0, SRC_PAD - N_SRC))
    as2_sc = jnp.pad(as2[:N_SRC], (0, SRC_PAD - N_SRC))
    ad1_sc = jnp.pad(ad1[:N_DST], (0, AD_PAD - N_DST))

    g1 = jnp.maximum(jnp.max(as1[:N_SRC]) + jnp.max(ad1[:N_DST]), 0.0)
    acc1, den1 = _sc_edge(hs1, as1_sc, ad1_sc, src_p, dst_p,
                          jnp.full((16,), g1, f32))

    ad2 = _postmv(acc1, den1, p1["bias"].reshape(1, HID), w2d,
                  jnp.zeros((1, 1), f32)).reshape(OUT_ROWS)
    ad2_sc = jnp.pad(ad2[:N_DST], (0, AD_PAD - N_DST))
    g2 = jnp.maximum(jnp.max(as2[:N_SRC]) + jnp.max(ad2[:N_DST]), 0.0)
    acc2, den2 = _sc_edge(hs2, as2_sc, ad2_sc, src_p, dst_p,
                          jnp.full((16,), g2, f32))

    out = _postmv(acc2, den2, p2["bias"].reshape(1, HID),
                  params["out_W"][:, 0].reshape(1, HID),
                  params["out_b"].reshape(1, 1)).reshape(OUT_ROWS)
    return out[:N_DST]
